# Initial kernel scaffold; baseline (speedup 1.0000x reference)
import jax
import jax.numpy as jnp
from jax.experimental import pallas as pl


def kernel(nbr_fea, ext_fea, W_value, b_value, W_off, b_off, W_attn, b_attn, W_out, b_out):
    raise NotImplementedError("write your pallas kernel here")


# speedup vs baseline: 263.7787x; 263.7787x over previous
import functools

import jax
import jax.numpy as jnp
import numpy as np
from jax import lax
from jax.experimental import pallas as pl
from jax.experimental.pallas import tpu as pltpu
from jax.experimental.pallas import tpu_sc as plsc

B = 2
H = 224
W = 224
LEN = H * W
D = 128
NH = 8
P = 4
DH = D // NH
T = 896
NBLK = LEN // T
GRID = B * NBLK
NGROUP = B * LEN * NH
NW = 32
GPW = NGROUP // NW
CG = 64
NCHUNK = GPW // CG
IDX_ROWS = NGROUP * DH // 128

_f32 = jnp.float32
_i32 = jnp.int32


def _selection_mats():
  sx = np.zeros((NH * P * 2, 128), np.float32)
  sy = np.zeros((NH * P * 2, 128), np.float32)
  sa = np.zeros((NH * P, 128), np.float32)
  for h in range(NH):
    for p in range(P):
      for c in range(4):
        tgt = h * 16 + p * 4 + c
        sx[(h * P + p) * 2 + 0, tgt] = 1.0
        sy[(h * P + p) * 2 + 1, tgt] = 1.0
        sa[h * P + p, tgt] = 1.0
  s4 = np.kron(np.eye(NH, dtype=np.float32), np.ones((P, P), np.float32))
  return jnp.asarray(sx), jnp.asarray(sy), jnp.asarray(sa), jnp.asarray(s4)


def _prep_body(nbr_ref, ext_ref, wv_ref, bv_ref, wox_ref, box_ref,
               woy_ref, boy_ref, wa_ref, ba_ref, s4_ref, sa_ref,
               val_ref, idx_ref, wt_ref):
  i = pl.program_id(0)
  dn_t = (((0,), (0,)), ((), ()))
  dn_n = (((1,), (0,)), ((), ()))
  x = nbr_ref[0]
  q = ext_ref[0]
  val = lax.dot_general(x, wv_ref[...], dn_t, preferred_element_type=_f32)
  val_ref[...] = val + bv_ref[...]
  offx = 10.0 * jnp.tanh(
      lax.dot_general(q, wox_ref[...], dn_t, preferred_element_type=_f32)
      + box_ref[...])
  offy = 10.0 * jnp.tanh(
      lax.dot_general(q, woy_ref[...], dn_t, preferred_element_type=_f32)
      + boy_ref[...])
  la = lax.dot_general(q, wa_ref[...], dn_t, preferred_element_type=_f32)
  la = la + ba_ref[...]
  la = la - jnp.max(la, axis=-1, keepdims=True)
  e = jnp.exp(la)
  den = lax.dot_general(e, s4_ref[...], dn_n, preferred_element_type=_f32)
  attn = lax.dot_general(e / den, sa_ref[...], dn_n,
                         preferred_element_type=_f32)

  qx3 = lax.broadcasted_iota(_i32, (4, W, 128), 1)
  qr3 = lax.broadcasted_iota(_i32, (4, W, 128), 0)
  qx = qx3.reshape(T, 128).astype(_f32)
  qy = ((i % NBLK) * 4 + qr3.reshape(T, 128)).astype(_f32)

  lane = lax.broadcasted_iota(_i32, (T, 128), 1)
  cx = (lane % 2).astype(_f32)
  cy = ((lane % 4) // 2).astype(_f32)
  px = qx + offx
  py = qy + offy
  x0 = jnp.floor(px)
  y0 = jnp.floor(py)
  fx = px - x0
  fy = py - y0
  xi = x0 + cx
  yi = y0 + cy
  wx = cx * fx + (1.0 - cx) * (1.0 - fx)
  wy = cy * fy + (1.0 - cy) * (1.0 - fy)
  valid = ((xi >= 0.0) & (xi <= W - 1.0) & (yi >= 0.0) & (yi <= H - 1.0))
  wt_ref[...] = wx * wy * attn * valid.astype(_f32)
  xi_i = jnp.clip(xi, 0.0, W - 1.0).astype(_i32)
  yi_i = jnp.clip(yi, 0.0, H - 1.0).astype(_i32)
  b = i // NBLK
  idx_ref[...] = (b * LEN + yi_i * W + xi_i) * NH + lane // 16


def _post_body(s_ref, wo_ref, bo_ref, out_ref):
  dn = (((1,), (0,)), ((), ()))
  out_ref[0] = (lax.dot_general(s_ref[...], wo_ref[...], dn,
                                preferred_element_type=_f32) + bo_ref[...])


def _sc_sample(table, idx2d, wtflat):
  mesh = plsc.VectorSubcoreMesh(core_axis_name="c", subcore_axis_name="s")

  @functools.partial(
      pl.kernel, mesh=mesh,
      compiler_params=pltpu.CompilerParams(use_tc_tiling_on_sc=False),
      out_type=jax.ShapeDtypeStruct((NGROUP, DH), _f32),
      scratch_types=[
          pltpu.VMEM((CG * DH // 128, 128), _i32),
          pltpu.VMEM((CG, DH), _f32),
          pltpu.VMEM((CG * DH, DH), _f32),
          pltpu.VMEM((CG, DH), _f32),
          pltpu.SemaphoreType.DMA,
      ],
  )
  def k(table_hbm, idx_hbm, wt_hbm, out_hbm, idx_v, wt_v, rows_v, out_v, sem):
    wid = lax.axis_index("s") * 2 + lax.axis_index("c")
    kslices = CG * DH // 128

    def chunk_body(ci, _):
      gbase = pl.multiple_of(wid * GPW + ci * CG, 64)
      pltpu.sync_copy(
          idx_hbm.at[pl.ds(pl.multiple_of(gbase * DH // 128, 8), kslices)],
          idx_v)
      pltpu.sync_copy(wt_hbm.at[pl.ds(gbase, CG)], wt_v)
      handles = []
      for s in range(kslices):
        handles.append(pltpu.async_copy(
            table_hbm.at[idx_v.at[s]],
            rows_v.at[pl.ds(s * 128, 128)], sem))
      for hd in handles:
        hd.wait()

      def group_body(g, _):
        wb = g * DH
        w_vec = wt_v[g]
        acc = jnp.zeros((DH,), _f32)
        for j in range(DH):
          wj = lax.gather(
              w_vec, jnp.full((DH, 1), j, _i32),
              lax.GatherDimensionNumbers(offset_dims=(),
                                         collapsed_slice_dims=(0,),
                                         start_index_map=(0,)),
              slice_sizes=(1,),
              mode=lax.GatherScatterMode.PROMISE_IN_BOUNDS)
          acc = acc + wj * rows_v[wb + j]
        out_v[g] = acc
        return 0

      lax.fori_loop(0, CG, group_body, 0)
      pltpu.sync_copy(out_v, out_hbm.at[pl.ds(gbase, CG)])
      return 0

    lax.fori_loop(0, NCHUNK, chunk_body, 0)

  return k(table, idx2d, wtflat)


def kernel(nbr_fea, ext_fea, W_value, b_value, W_off, b_off, W_attn, b_attn,
           W_out, b_out):
  sx, sy, sa, s4 = _selection_mats()
  wox = W_off @ sx
  box = (b_off @ sx).reshape(1, 128)
  woy = W_off @ sy
  boy = (b_off @ sy).reshape(1, 128)

  nbr = nbr_fea.reshape(B, D, LEN)
  ext = ext_fea.reshape(B, D, LEN)
  full = lambda s: pl.BlockSpec(s, lambda i: (0,) * len(s))
  val, idx, wt = pl.pallas_call(
      _prep_body,
      grid=(GRID,),
      in_specs=[
          pl.BlockSpec((1, D, T), lambda i: (i // NBLK, 0, i % NBLK)),
          pl.BlockSpec((1, D, T), lambda i: (i // NBLK, 0, i % NBLK)),
          full((D, D)), full((1, D)),
          full((D, 128)), full((1, 128)),
          full((D, 128)), full((1, 128)),
          full((D, NH * P)), full((1, NH * P)),
          full((NH * P, NH * P)), full((NH * P, 128)),
      ],
      out_specs=[
          pl.BlockSpec((T, D), lambda i: (i, 0)),
          pl.BlockSpec((T, 128), lambda i: (i, 0)),
          pl.BlockSpec((T, 128), lambda i: (i, 0)),
      ],
      out_shape=[
          jax.ShapeDtypeStruct((B * LEN, D), _f32),
          jax.ShapeDtypeStruct((B * LEN, 128), _i32),
          jax.ShapeDtypeStruct((B * LEN, 128), _f32),
      ],
  )(nbr, ext, W_value, b_value.reshape(1, D), wox, box, woy, boy,
    W_attn, b_attn.reshape(1, NH * P), s4, sa)

  sampled = _sc_sample(val.reshape(NGROUP, DH),
                       idx.reshape(IDX_ROWS, 128),
                       wt.reshape(NGROUP, DH))

  out = pl.pallas_call(
      _post_body,
      grid=(GRID,),
      in_specs=[
          pl.BlockSpec((T, D), lambda i: (i, 0)),
          full((D, D)), full((1, D)),
      ],
      out_specs=pl.BlockSpec((1, T, D), lambda i: (i // NBLK, i % NBLK, 0)),
      out_shape=jax.ShapeDtypeStruct((B, LEN, D), _f32),
  )(sampled.reshape(B * LEN, D), W_out, b_out.reshape(1, D))
  return out


# speedup vs baseline: 450.1290x; 1.7065x over previous
import functools

import jax
import jax.numpy as jnp
import numpy as np
from jax import lax
from jax.experimental import pallas as pl
from jax.experimental.pallas import tpu as pltpu
from jax.experimental.pallas import tpu_sc as plsc

B = 2
H = 224
W = 224
LEN = H * W
D = 128
NH = 8
P = 4
DH = D // NH
T = 896
NBLK = LEN // T
GRID = B * NBLK
NGROUP = B * LEN * NH
NW = 32
GPW = NGROUP // NW
CG = 128
NCHUNK = GPW // CG
KS = CG * DH // 128
IDX_ROWS = NGROUP * DH // 128

_f32 = jnp.float32
_i32 = jnp.int32


def _selection_mats():
  sx = np.zeros((NH * P * 2, 128), np.float32)
  sy = np.zeros((NH * P * 2, 128), np.float32)
  sa = np.zeros((NH * P, 128), np.float32)
  for h in range(NH):
    for p in range(P):
      for c in range(4):
        tgt = h * 16 + p * 4 + c
        sx[(h * P + p) * 2 + 0, tgt] = 1.0
        sy[(h * P + p) * 2 + 1, tgt] = 1.0
        sa[h * P + p, tgt] = 1.0
  s4 = np.kron(np.eye(NH, dtype=np.float32), np.ones((P, P), np.float32))
  return jnp.asarray(sx), jnp.asarray(sy), jnp.asarray(sa), jnp.asarray(s4)


def _prep_body(nbr_ref, ext_ref, wv_ref, bv_ref, wox_ref, box_ref,
               woy_ref, boy_ref, wa_ref, ba_ref, s4_ref, sa_ref,
               val_ref, idx_ref, wt_ref):
  i = pl.program_id(0)
  dn_t = (((0,), (0,)), ((), ()))
  dn_n = (((1,), (0,)), ((), ()))
  x = nbr_ref[0]
  q = ext_ref[0]
  val = lax.dot_general(x, wv_ref[...], dn_t, preferred_element_type=_f32)
  val_ref[...] = val + bv_ref[...]
  offx = 10.0 * jnp.tanh(
      lax.dot_general(q, wox_ref[...], dn_t, preferred_element_type=_f32)
      + box_ref[...])
  offy = 10.0 * jnp.tanh(
      lax.dot_general(q, woy_ref[...], dn_t, preferred_element_type=_f32)
      + boy_ref[...])
  la = lax.dot_general(q, wa_ref[...], dn_t, preferred_element_type=_f32)
  la = la + ba_ref[...]
  la = la - jnp.max(la, axis=-1, keepdims=True)
  e = jnp.exp(la)
  den = lax.dot_general(e, s4_ref[...], dn_n, preferred_element_type=_f32)
  attn = lax.dot_general(e / den, sa_ref[...], dn_n,
                         preferred_element_type=_f32)

  qx3 = lax.broadcasted_iota(_i32, (4, W, 128), 1)
  qr3 = lax.broadcasted_iota(_i32, (4, W, 128), 0)
  qx = qx3.reshape(T, 128).astype(_f32)
  qy = ((i % NBLK) * 4 + qr3.reshape(T, 128)).astype(_f32)

  lane = lax.broadcasted_iota(_i32, (T, 128), 1)
  cx = (lane % 2).astype(_f32)
  cy = ((lane % 4) // 2).astype(_f32)
  px = qx + offx
  py = qy + offy
  x0 = jnp.floor(px)
  y0 = jnp.floor(py)
  fx = px - x0
  fy = py - y0
  xi = x0 + cx
  yi = y0 + cy
  wx = cx * fx + (1.0 - cx) * (1.0 - fx)
  wy = cy * fy + (1.0 - cy) * (1.0 - fy)
  valid = ((xi >= 0.0) & (xi <= W - 1.0) & (yi >= 0.0) & (yi <= H - 1.0))
  wt_ref[...] = wx * wy * attn * valid.astype(_f32)
  xi_i = jnp.clip(xi, 0.0, W - 1.0).astype(_i32)
  yi_i = jnp.clip(yi, 0.0, H - 1.0).astype(_i32)
  b = i // NBLK
  idx_ref[...] = (b * LEN + yi_i * W + xi_i) * NH + lane // 16


def _post_body(s_ref, wo_ref, bo_ref, out_ref):
  dn = (((1,), (0,)), ((), ()))
  out_ref[0] = (lax.dot_general(s_ref[...], wo_ref[...], dn,
                                preferred_element_type=_f32) + bo_ref[...])


def _sc_sample(table, idx2d, wtflat):
  mesh = plsc.VectorSubcoreMesh(core_axis_name="c", subcore_axis_name="s")

  @functools.partial(
      pl.kernel, mesh=mesh,
      compiler_params=pltpu.CompilerParams(use_tc_tiling_on_sc=False),
      out_type=jax.ShapeDtypeStruct((NGROUP, DH), _f32),
      scratch_types=[
          pltpu.VMEM((KS, 128), _i32),
          pltpu.VMEM((KS, 128), _i32),
          pltpu.VMEM((CG, DH), _f32),
          pltpu.VMEM((CG, DH), _f32),
          pltpu.VMEM((CG * DH, DH), _f32),
          pltpu.VMEM((CG * DH, DH), _f32),
          pltpu.VMEM((CG, DH), _f32),
          pltpu.SemaphoreType.DMA,
          pltpu.SemaphoreType.DMA,
      ],
  )
  def k(table_hbm, idx_hbm, wt_hbm, out_hbm, idx_a, idx_b, wt_a, wt_b,
        rows_a, rows_b, out_v, sem_a, sem_b):
    wid = lax.axis_index("s") * 2 + lax.axis_index("c")

    def load_and_fire(ci, idx_v, rows_v, sem):
      gbase = pl.multiple_of(wid * GPW + ci * CG, CG)
      pltpu.sync_copy(
          idx_hbm.at[pl.ds(pl.multiple_of(gbase * DH // 128, 8), KS)], idx_v)
      for s in range(KS):
        pltpu.async_copy(table_hbm.at[idx_v.at[s]],
                         rows_v.at[pl.ds(s * 128, 128)], sem)

    def compute(ci, wt_v, rows_v, sem):
      gbase = pl.multiple_of(wid * GPW + ci * CG, CG)
      pltpu.sync_copy(wt_hbm.at[pl.ds(gbase, CG)], wt_v)
      pltpu.make_async_copy(table_hbm.at[pl.ds(0, CG * DH)], rows_v,
                            sem).wait()

      def group_body(g, _):
        wb = g * DH
        w_vec = wt_v[g]
        acc = jnp.zeros((DH,), _f32)
        for j in range(DH):
          wj = lax.gather(
              w_vec, jnp.full((DH, 1), j, _i32),
              lax.GatherDimensionNumbers(offset_dims=(),
                                         collapsed_slice_dims=(0,),
                                         start_index_map=(0,)),
              slice_sizes=(1,),
              mode=lax.GatherScatterMode.PROMISE_IN_BOUNDS)
          acc = acc + wj * rows_v[wb + j]
        out_v[g] = acc
        return 0

      lax.fori_loop(0, CG, group_body, 0)
      pltpu.sync_copy(out_v, out_hbm.at[pl.ds(gbase, CG)])

    load_and_fire(0, idx_a, rows_a, sem_a)

    def pair_body(t, _):
      load_and_fire(2 * t + 1, idx_b, rows_b, sem_b)
      compute(2 * t, wt_a, rows_a, sem_a)

      @pl.when(t < NCHUNK // 2 - 1)
      def _():
        load_and_fire(2 * t + 2, idx_a, rows_a, sem_a)

      compute(2 * t + 1, wt_b, rows_b, sem_b)
      return 0

    lax.fori_loop(0, NCHUNK // 2, pair_body, 0)

  return k(table, idx2d, wtflat)


def kernel(nbr_fea, ext_fea, W_value, b_value, W_off, b_off, W_attn, b_attn,
           W_out, b_out):
  sx, sy, sa, s4 = _selection_mats()
  wox = W_off @ sx
  box = (b_off @ sx).reshape(1, 128)
  woy = W_off @ sy
  boy = (b_off @ sy).reshape(1, 128)

  nbr = nbr_fea.reshape(B, D, LEN)
  ext = ext_fea.reshape(B, D, LEN)
  full = lambda s: pl.BlockSpec(s, lambda i: (0,) * len(s))
  val, idx, wt = pl.pallas_call(
      _prep_body,
      grid=(GRID,),
      in_specs=[
          pl.BlockSpec((1, D, T), lambda i: (i // NBLK, 0, i % NBLK)),
          pl.BlockSpec((1, D, T), lambda i: (i // NBLK, 0, i % NBLK)),
          full((D, D)), full((1, D)),
          full((D, 128)), full((1, 128)),
          full((D, 128)), full((1, 128)),
          full((D, NH * P)), full((1, NH * P)),
          full((NH * P, NH * P)), full((NH * P, 128)),
      ],
      out_specs=[
          pl.BlockSpec((T, D), lambda i: (i, 0)),
          pl.BlockSpec((T, 128), lambda i: (i, 0)),
          pl.BlockSpec((T, 128), lambda i: (i, 0)),
      ],
      out_shape=[
          jax.ShapeDtypeStruct((B * LEN, D), _f32),
          jax.ShapeDtypeStruct((B * LEN, 128), _i32),
          jax.ShapeDtypeStruct((B * LEN, 128), _f32),
      ],
  )(nbr, ext, W_value, b_value.reshape(1, D), wox, box, woy, boy,
    W_attn, b_attn.reshape(1, NH * P), s4, sa)

  sampled = _sc_sample(val.reshape(NGROUP, DH),
                       idx.reshape(IDX_ROWS, 128),
                       wt.reshape(NGROUP, DH))

  out = pl.pallas_call(
      _post_body,
      grid=(GRID,),
      in_specs=[
          pl.BlockSpec((T, D), lambda i: (i, 0)),
          full((D, D)), full((1, D)),
      ],
      out_specs=pl.BlockSpec((1, T, D), lambda i: (i // NBLK, i % NBLK, 0)),
      out_shape=jax.ShapeDtypeStruct((B, LEN, D), _f32),
  )(sampled.reshape(B * LEN, D), W_out, b_out.reshape(1, D))
  return out


# speedup vs baseline: 633.8393x; 1.4081x over previous
import functools

import jax
import jax.numpy as jnp
import numpy as np
from jax import lax
from jax.experimental import pallas as pl
from jax.experimental.pallas import tpu as pltpu
from jax.experimental.pallas import tpu_sc as plsc

B = 2
H = 224
W = 224
LEN = H * W
D = 128
NH = 8
P = 4
DH = D // NH
T = 896
NBLK = LEN // T
GRID = B * NBLK
NGROUP = B * LEN * NH
NW = 32
GPW = NGROUP // NW
CG = 128
NCHUNK = GPW // CG
KS = CG * DH // 128
IDX_ROWS = NGROUP * DH // 128

_f32 = jnp.float32
_i32 = jnp.int32


def _selection_mats():
  sx = np.zeros((NH * P * 2, 128), np.float32)
  sy = np.zeros((NH * P * 2, 128), np.float32)
  sa = np.zeros((NH * P, 128), np.float32)
  for h in range(NH):
    for p in range(P):
      for c in range(4):
        tgt = h * 16 + p * 4 + c
        sx[(h * P + p) * 2 + 0, tgt] = 1.0
        sy[(h * P + p) * 2 + 1, tgt] = 1.0
        sa[h * P + p, tgt] = 1.0
  s4 = np.kron(np.eye(NH, dtype=np.float32), np.ones((P, P), np.float32))
  return jnp.asarray(sx), jnp.asarray(sy), jnp.asarray(sa), jnp.asarray(s4)


def _prep_body(nbr_ref, ext_ref, wv_ref, bv_ref, wox_ref, box_ref,
               woy_ref, boy_ref, wa_ref, ba_ref, s4_ref, sa_ref,
               val_ref, idx_ref, wt_ref):
  i = pl.program_id(0)
  dn_t = (((0,), (0,)), ((), ()))
  dn_n = (((1,), (0,)), ((), ()))
  x = nbr_ref[0]
  q = ext_ref[0]
  val = lax.dot_general(x, wv_ref[...], dn_t, preferred_element_type=_f32)
  val_ref[...] = val + bv_ref[...]
  offx = 10.0 * jnp.tanh(
      lax.dot_general(q, wox_ref[...], dn_t, preferred_element_type=_f32)
      + box_ref[...])
  offy = 10.0 * jnp.tanh(
      lax.dot_general(q, woy_ref[...], dn_t, preferred_element_type=_f32)
      + boy_ref[...])
  la = lax.dot_general(q, wa_ref[...], dn_t, preferred_element_type=_f32)
  la = la + ba_ref[...]
  la = la - jnp.max(la, axis=-1, keepdims=True)
  e = jnp.exp(la)
  den = lax.dot_general(e, s4_ref[...], dn_n, preferred_element_type=_f32)
  attn = lax.dot_general(e / den, sa_ref[...], dn_n,
                         preferred_element_type=_f32)

  qx3 = lax.broadcasted_iota(_i32, (4, W, 128), 1)
  qr3 = lax.broadcasted_iota(_i32, (4, W, 128), 0)
  qx = qx3.reshape(T, 128).astype(_f32)
  qy = ((i % NBLK) * 4 + qr3.reshape(T, 128)).astype(_f32)

  lane = lax.broadcasted_iota(_i32, (T, 128), 1)
  cx = (lane % 2).astype(_f32)
  cy = ((lane % 4) // 2).astype(_f32)
  px = qx + offx
  py = qy + offy
  x0 = jnp.floor(px)
  y0 = jnp.floor(py)
  fx = px - x0
  fy = py - y0
  xi = x0 + cx
  yi = y0 + cy
  wx = cx * fx + (1.0 - cx) * (1.0 - fx)
  wy = cy * fy + (1.0 - cy) * (1.0 - fy)
  valid = ((xi >= 0.0) & (xi <= W - 1.0) & (yi >= 0.0) & (yi <= H - 1.0))
  wt_ref[...] = wx * wy * attn * valid.astype(_f32)
  xi_i = jnp.clip(xi, 0.0, W - 1.0).astype(_i32)
  yi_i = jnp.clip(yi, 0.0, H - 1.0).astype(_i32)
  b = i // NBLK
  idx_ref[...] = (b * LEN + yi_i * W + xi_i) * NH + lane // 16


def _post_body(s_ref, wo_ref, bo_ref, out_ref):
  dn = (((1,), (0,)), ((), ()))
  out_ref[0] = (lax.dot_general(s_ref[...], wo_ref[...], dn,
                                preferred_element_type=_f32) + bo_ref[...])


def _sc_sample(table, idx2d, wtflat):
  mesh = plsc.VectorSubcoreMesh(core_axis_name="c", subcore_axis_name="s")

  @functools.partial(
      pl.kernel, mesh=mesh,
      compiler_params=pltpu.CompilerParams(use_tc_tiling_on_sc=False),
      out_type=jax.ShapeDtypeStruct((NGROUP, DH), _f32),
      scratch_types=[
          pltpu.VMEM((KS, 128), _i32),
          pltpu.VMEM((KS, 128), _i32),
          pltpu.VMEM((CG, DH), _f32),
          pltpu.VMEM((CG, DH), _f32),
          pltpu.VMEM((CG * DH, DH), _f32),
          pltpu.VMEM((CG * DH, DH), _f32),
          pltpu.VMEM((CG, DH), _f32),
          pltpu.VMEM((CG, DH), _f32),
          pltpu.SemaphoreType.DMA,
          pltpu.SemaphoreType.DMA,
          pltpu.SemaphoreType.DMA,
          pltpu.SemaphoreType.DMA,
      ],
  )
  def k(table_hbm, idx_hbm, wt_hbm, out_hbm, idx_a, idx_b, wt_a, wt_b,
        rows_a, rows_b, out_a, out_b, sem_a, sem_b, sem_oa, sem_ob):
    wid = lax.axis_index("s") * 2 + lax.axis_index("c")

    def load_and_fire(ci, idx_v, wt_v, rows_v, sem):
      gbase = pl.multiple_of(wid * GPW + ci * CG, CG)
      pltpu.async_copy(wt_hbm.at[pl.ds(gbase, CG)], wt_v, sem)
      pltpu.sync_copy(
          idx_hbm.at[pl.ds(pl.multiple_of(gbase * DH // 128, 8), KS)], idx_v)
      for s in range(KS):
        pltpu.async_copy(table_hbm.at[idx_v.at[s]],
                         rows_v.at[pl.ds(s * 128, 128)], sem)

    def compute(ci, wt_v, rows_v, sem, out_v, sem_o):
      gbase = pl.multiple_of(wid * GPW + ci * CG, CG)
      pltpu.make_async_copy(table_hbm.at[pl.ds(0, CG * DH)], rows_v,
                            sem).wait()
      pltpu.make_async_copy(wt_hbm.at[pl.ds(0, CG)], wt_v, sem).wait()

      @pl.when(ci >= 2)
      def _():
        pltpu.make_async_copy(out_v, out_hbm.at[pl.ds(0, CG)], sem_o).wait()

      @plsc.parallel_loop(0, CG, unroll=4)
      def group_body(g):
        wb = g * DH
        w_vec = wt_v[g]
        acc = jnp.zeros((DH,), _f32)
        for j in range(DH):
          wj = lax.gather(
              w_vec, jnp.full((DH, 1), j, _i32),
              lax.GatherDimensionNumbers(offset_dims=(),
                                         collapsed_slice_dims=(0,),
                                         start_index_map=(0,)),
              slice_sizes=(1,),
              mode=lax.GatherScatterMode.PROMISE_IN_BOUNDS)
          acc = acc + wj * rows_v[wb + j]
        out_v[g] = acc

      pltpu.async_copy(out_v, out_hbm.at[pl.ds(gbase, CG)], sem_o)

    load_and_fire(0, idx_a, wt_a, rows_a, sem_a)

    def pair_body(t, _):
      load_and_fire(2 * t + 1, idx_b, wt_b, rows_b, sem_b)
      compute(2 * t, wt_a, rows_a, sem_a, out_a, sem_oa)

      @pl.when(t < NCHUNK // 2 - 1)
      def _():
        load_and_fire(2 * t + 2, idx_a, wt_a, rows_a, sem_a)

      compute(2 * t + 1, wt_b, rows_b, sem_b, out_b, sem_ob)
      return 0

    lax.fori_loop(0, NCHUNK // 2, pair_body, 0)
    pltpu.make_async_copy(out_a, out_hbm.at[pl.ds(0, CG)], sem_oa).wait()
    pltpu.make_async_copy(out_b, out_hbm.at[pl.ds(0, CG)], sem_ob).wait()

  return k(table, idx2d, wtflat)


def kernel(nbr_fea, ext_fea, W_value, b_value, W_off, b_off, W_attn, b_attn,
           W_out, b_out):
  sx, sy, sa, s4 = _selection_mats()
  wox = W_off @ sx
  box = (b_off @ sx).reshape(1, 128)
  woy = W_off @ sy
  boy = (b_off @ sy).reshape(1, 128)

  nbr = nbr_fea.reshape(B, D, LEN)
  ext = ext_fea.reshape(B, D, LEN)
  full = lambda s: pl.BlockSpec(s, lambda i: (0,) * len(s))
  val, idx, wt = pl.pallas_call(
      _prep_body,
      grid=(GRID,),
      in_specs=[
          pl.BlockSpec((1, D, T), lambda i: (i // NBLK, 0, i % NBLK)),
          pl.BlockSpec((1, D, T), lambda i: (i // NBLK, 0, i % NBLK)),
          full((D, D)), full((1, D)),
          full((D, 128)), full((1, 128)),
          full((D, 128)), full((1, 128)),
          full((D, NH * P)), full((1, NH * P)),
          full((NH * P, NH * P)), full((NH * P, 128)),
      ],
      out_specs=[
          pl.BlockSpec((T, D), lambda i: (i, 0)),
          pl.BlockSpec((T, 128), lambda i: (i, 0)),
          pl.BlockSpec((T, 128), lambda i: (i, 0)),
      ],
      out_shape=[
          jax.ShapeDtypeStruct((B * LEN, D), _f32),
          jax.ShapeDtypeStruct((B * LEN, 128), _i32),
          jax.ShapeDtypeStruct((B * LEN, 128), _f32),
      ],
  )(nbr, ext, W_value, b_value.reshape(1, D), wox, box, woy, boy,
    W_attn, b_attn.reshape(1, NH * P), s4, sa)

  sampled = _sc_sample(val.reshape(NGROUP, DH),
                       idx.reshape(IDX_ROWS, 128),
                       wt.reshape(NGROUP, DH))

  out = pl.pallas_call(
      _post_body,
      grid=(GRID,),
      in_specs=[
          pl.BlockSpec((T, D), lambda i: (i, 0)),
          full((D, D)), full((1, D)),
      ],
      out_specs=pl.BlockSpec((1, T, D), lambda i: (i // NBLK, i % NBLK, 0)),
      out_shape=jax.ShapeDtypeStruct((B, LEN, D), _f32),
  )(sampled.reshape(B * LEN, D), W_out, b_out.reshape(1, D))
  return out


# speedup vs baseline: 650.1666x; 1.0258x over previous
import functools

import jax
import jax.numpy as jnp
import numpy as np
from jax import lax
from jax.experimental import pallas as pl
from jax.experimental.pallas import tpu as pltpu
from jax.experimental.pallas import tpu_sc as plsc

B = 2
H = 224
W = 224
LEN = H * W
D = 128
NH = 8
P = 4
DH = D // NH
T = 896
NBLK = LEN // T
GRID = B * NBLK
NGROUP = B * LEN * NH
NW = 32
GPW = NGROUP // NW
CG = 128
NCHUNK = GPW // CG
KS = CG * DH // 128
IDX_ROWS = NGROUP * DH // 128

_f32 = jnp.float32
_i32 = jnp.int32


def _selection_mats():
  sx = np.zeros((NH * P * 2, 128), np.float32)
  sy = np.zeros((NH * P * 2, 128), np.float32)
  sa = np.zeros((NH * P, 128), np.float32)
  for h in range(NH):
    for p in range(P):
      for c in range(4):
        tgt = h * 16 + p * 4 + c
        sx[(h * P + p) * 2 + 0, tgt] = 1.0
        sy[(h * P + p) * 2 + 1, tgt] = 1.0
        sa[h * P + p, tgt] = 1.0
  s4 = np.kron(np.eye(NH, dtype=np.float32), np.ones((P, P), np.float32))
  return jnp.asarray(sx), jnp.asarray(sy), jnp.asarray(sa), jnp.asarray(s4)


def _prep_body(nbr_ref, ext_ref, wv_ref, bv_ref, wox_ref, box_ref,
               woy_ref, boy_ref, wa_ref, ba_ref, s4_ref, sa_ref,
               val_ref, idx_ref, wt_ref):
  i = pl.program_id(0)
  dn_t = (((0,), (0,)), ((), ()))
  dn_n = (((1,), (0,)), ((), ()))
  x = nbr_ref[0]
  q = ext_ref[0]
  val = lax.dot_general(x, wv_ref[...], dn_t, preferred_element_type=_f32)
  val_ref[...] = val + bv_ref[...]
  offx = 10.0 * jnp.tanh(
      lax.dot_general(q, wox_ref[...], dn_t, preferred_element_type=_f32)
      + box_ref[...])
  offy = 10.0 * jnp.tanh(
      lax.dot_general(q, woy_ref[...], dn_t, preferred_element_type=_f32)
      + boy_ref[...])
  la = lax.dot_general(q, wa_ref[...], dn_t, preferred_element_type=_f32)
  la = la + ba_ref[...]
  la = la - jnp.max(la, axis=-1, keepdims=True)
  e = jnp.exp(la)
  den = lax.dot_general(e, s4_ref[...], dn_n, preferred_element_type=_f32)
  attn = lax.dot_general(e / den, sa_ref[...], dn_n,
                         preferred_element_type=_f32)

  qx3 = lax.broadcasted_iota(_i32, (4, W, 128), 1)
  qr3 = lax.broadcasted_iota(_i32, (4, W, 128), 0)
  qx = qx3.reshape(T, 128).astype(_f32)
  qy = ((i % NBLK) * 4 + qr3.reshape(T, 128)).astype(_f32)

  lane = lax.broadcasted_iota(_i32, (T, 128), 1)
  cx = (lane % 2).astype(_f32)
  cy = ((lane % 4) // 2).astype(_f32)
  px = qx + offx
  py = qy + offy
  x0 = jnp.floor(px)
  y0 = jnp.floor(py)
  fx = px - x0
  fy = py - y0
  xi = x0 + cx
  yi = y0 + cy
  wx = cx * fx + (1.0 - cx) * (1.0 - fx)
  wy = cy * fy + (1.0 - cy) * (1.0 - fy)
  valid = ((xi >= 0.0) & (xi <= W - 1.0) & (yi >= 0.0) & (yi <= H - 1.0))
  wt_ref[...] = wx * wy * attn * valid.astype(_f32)
  xi_i = jnp.clip(xi, 0.0, W - 1.0).astype(_i32)
  yi_i = jnp.clip(yi, 0.0, H - 1.0).astype(_i32)
  b = i // NBLK
  idx_ref[...] = (b * LEN + yi_i * W + xi_i) * NH + lane // 16


def _post_body(s_ref, wo_ref, bo_ref, out_ref):
  dn = (((1,), (0,)), ((), ()))
  out_ref[0] = (lax.dot_general(s_ref[...], wo_ref[...], dn,
                                preferred_element_type=_f32) + bo_ref[...])


def _sc_sample(table, idx2d, wtflat):
  mesh = plsc.VectorSubcoreMesh(core_axis_name="c", subcore_axis_name="s")

  @functools.partial(
      pl.kernel, mesh=mesh,
      compiler_params=pltpu.CompilerParams(use_tc_tiling_on_sc=False),
      out_type=jax.ShapeDtypeStruct((NGROUP, DH), _f32),
      scratch_types=[
          pltpu.VMEM((KS, 128), _i32),
          pltpu.VMEM((KS, 128), _i32),
          pltpu.VMEM((CG, DH), _f32),
          pltpu.VMEM((CG, DH), _f32),
          pltpu.VMEM((CG * DH, DH), _f32),
          pltpu.VMEM((CG * DH, DH), _f32),
          pltpu.VMEM((CG, DH), _f32),
          pltpu.VMEM((CG, DH), _f32),
          pltpu.SemaphoreType.DMA,
          pltpu.SemaphoreType.DMA,
          pltpu.SemaphoreType.DMA,
          pltpu.SemaphoreType.DMA,
      ],
  )
  def k(table_hbm, idx_hbm, wt_hbm, out_hbm, idx_a, idx_b, wt_a, wt_b,
        rows_a, rows_b, out_a, out_b, sem_a, sem_b, sem_oa, sem_ob):
    wid = lax.axis_index("s") * 2 + lax.axis_index("c")

    def fire_idx(ci, idx_v, sem):
      gbase = pl.multiple_of(wid * GPW + ci * CG, CG)
      pltpu.async_copy(
          idx_hbm.at[pl.ds(pl.multiple_of(gbase * DH // 128, 8), KS)], idx_v,
          sem)

    def fire_wt(ci, wt_v, sem):
      gbase = pl.multiple_of(wid * GPW + ci * CG, CG)
      pltpu.async_copy(wt_hbm.at[pl.ds(gbase, CG)], wt_v, sem)

    def fire_gathers(idx_v, rows_v, sem):
      pltpu.make_async_copy(
          idx_hbm.at[pl.ds(0, KS)], idx_v, sem).wait()
      for s in range(KS):
        pltpu.async_copy(table_hbm.at[idx_v.at[s]],
                         rows_v.at[pl.ds(s * 128, 128)], sem)

    def compute(ci, idx_v, wt_v, rows_v, sem, out_v, sem_o):
      gbase = pl.multiple_of(wid * GPW + ci * CG, CG)
      pltpu.make_async_copy(table_hbm.at[pl.ds(0, CG * DH)], rows_v,
                            sem).wait()
      pltpu.make_async_copy(wt_hbm.at[pl.ds(0, CG)], wt_v, sem).wait()

      @pl.when(ci + 2 < NCHUNK)
      def _():
        fire_idx(ci + 2, idx_v, sem)

      @pl.when(ci >= 2)
      def _():
        pltpu.make_async_copy(out_v, out_hbm.at[pl.ds(0, CG)], sem_o).wait()

      @plsc.parallel_loop(0, CG, unroll=8)
      def group_body(g):
        wb = g * DH
        w_vec = wt_v[g]
        acc = jnp.zeros((DH,), _f32)
        for j in range(DH):
          wj = lax.gather(
              w_vec, jnp.full((DH, 1), j, _i32),
              lax.GatherDimensionNumbers(offset_dims=(),
                                         collapsed_slice_dims=(0,),
                                         start_index_map=(0,)),
              slice_sizes=(1,),
              mode=lax.GatherScatterMode.PROMISE_IN_BOUNDS)
          acc = acc + wj * rows_v[wb + j]
        out_v[g] = acc

      @pl.when(ci + 2 < NCHUNK)
      def _():
        fire_wt(ci + 2, wt_v, sem)

      pltpu.async_copy(out_v, out_hbm.at[pl.ds(gbase, CG)], sem_o)

    fire_idx(0, idx_a, sem_a)
    fire_wt(0, wt_a, sem_a)
    fire_idx(1, idx_b, sem_b)
    fire_wt(1, wt_b, sem_b)
    fire_gathers(idx_a, rows_a, sem_a)

    def pair_body(t, _):
      fire_gathers(idx_b, rows_b, sem_b)
      compute(2 * t, idx_a, wt_a, rows_a, sem_a, out_a, sem_oa)

      @pl.when(t < NCHUNK // 2 - 1)
      def _():
        fire_gathers(idx_a, rows_a, sem_a)

      compute(2 * t + 1, idx_b, wt_b, rows_b, sem_b, out_b, sem_ob)
      return 0

    lax.fori_loop(0, NCHUNK // 2, pair_body, 0)
    pltpu.make_async_copy(out_a, out_hbm.at[pl.ds(0, CG)], sem_oa).wait()
    pltpu.make_async_copy(out_b, out_hbm.at[pl.ds(0, CG)], sem_ob).wait()

  return k(table, idx2d, wtflat)


def kernel(nbr_fea, ext_fea, W_value, b_value, W_off, b_off, W_attn, b_attn,
           W_out, b_out):
  sx, sy, sa, s4 = _selection_mats()
  wox = W_off @ sx
  box = (b_off @ sx).reshape(1, 128)
  woy = W_off @ sy
  boy = (b_off @ sy).reshape(1, 128)

  nbr = nbr_fea.reshape(B, D, LEN)
  ext = ext_fea.reshape(B, D, LEN)
  full = lambda s: pl.BlockSpec(s, lambda i: (0,) * len(s))
  val, idx, wt = pl.pallas_call(
      _prep_body,
      grid=(GRID,),
      in_specs=[
          pl.BlockSpec((1, D, T), lambda i: (i // NBLK, 0, i % NBLK)),
          pl.BlockSpec((1, D, T), lambda i: (i // NBLK, 0, i % NBLK)),
          full((D, D)), full((1, D)),
          full((D, 128)), full((1, 128)),
          full((D, 128)), full((1, 128)),
          full((D, NH * P)), full((1, NH * P)),
          full((NH * P, NH * P)), full((NH * P, 128)),
      ],
      out_specs=[
          pl.BlockSpec((T, D), lambda i: (i, 0)),
          pl.BlockSpec((T, 128), lambda i: (i, 0)),
          pl.BlockSpec((T, 128), lambda i: (i, 0)),
      ],
      out_shape=[
          jax.ShapeDtypeStruct((B * LEN, D), _f32),
          jax.ShapeDtypeStruct((B * LEN, 128), _i32),
          jax.ShapeDtypeStruct((B * LEN, 128), _f32),
      ],
  )(nbr, ext, W_value, b_value.reshape(1, D), wox, box, woy, boy,
    W_attn, b_attn.reshape(1, NH * P), s4, sa)

  sampled = _sc_sample(val.reshape(NGROUP, DH),
                       idx.reshape(IDX_ROWS, 128),
                       wt.reshape(NGROUP, DH))

  out = pl.pallas_call(
      _post_body,
      grid=(GRID,),
      in_specs=[
          pl.BlockSpec((T, D), lambda i: (i, 0)),
          full((D, D)), full((1, D)),
      ],
      out_specs=pl.BlockSpec((1, T, D), lambda i: (i // NBLK, i % NBLK, 0)),
      out_shape=jax.ShapeDtypeStruct((B, LEN, D), _f32),
  )(sampled.reshape(B * LEN, D), W_out, b_out.reshape(1, D))
  return out


# speedup vs baseline: 650.6498x; 1.0007x over previous
import functools

import jax
import jax.numpy as jnp
import numpy as np
from jax import lax
from jax.experimental import pallas as pl
from jax.experimental.pallas import tpu as pltpu
from jax.experimental.pallas import tpu_sc as plsc

B = 2
H = 224
W = 224
LEN = H * W
D = 128
NH = 8
P = 4
DH = D // NH
T = 896
NBLK = LEN // T
GRID = B * NBLK
NGROUP = B * LEN * NH
NW = 32
GPW = NGROUP // NW
CG = 128
NCHUNK = GPW // CG
KS = CG * DH // 128
IDX_ROWS = NGROUP * DH // 128

_f32 = jnp.float32
_i32 = jnp.int32


def _selection_mats():
  sx = np.zeros((NH * P * 2, 128), np.float32)
  sy = np.zeros((NH * P * 2, 128), np.float32)
  sa = np.zeros((NH * P, 128), np.float32)
  for h in range(NH):
    for p in range(P):
      for c in range(4):
        tgt = h * 16 + p * 4 + c
        sx[(h * P + p) * 2 + 0, tgt] = 1.0
        sy[(h * P + p) * 2 + 1, tgt] = 1.0
        sa[h * P + p, tgt] = 1.0
  s4 = np.kron(np.eye(NH, dtype=np.float32), np.ones((P, P), np.float32))
  return jnp.asarray(sx), jnp.asarray(sy), jnp.asarray(sa), jnp.asarray(s4)


def _prep_body(nbr_ref, ext_ref, wv_ref, bv_ref, wox_ref, box_ref,
               woy_ref, boy_ref, wa_ref, ba_ref, s4_ref, sa_ref,
               val_ref, idx_ref, wt_ref):
  i = pl.program_id(0)
  dn_t = (((0,), (0,)), ((), ()))
  dn_n = (((1,), (0,)), ((), ()))
  x = nbr_ref[0]
  q = ext_ref[0]
  val = lax.dot_general(x, wv_ref[...], dn_t, preferred_element_type=_f32)
  val_ref[...] = val + bv_ref[...]
  offx = 10.0 * jnp.tanh(
      lax.dot_general(q, wox_ref[...], dn_t, preferred_element_type=_f32)
      + box_ref[...])
  offy = 10.0 * jnp.tanh(
      lax.dot_general(q, woy_ref[...], dn_t, preferred_element_type=_f32)
      + boy_ref[...])
  la = lax.dot_general(q, wa_ref[...], dn_t, preferred_element_type=_f32)
  la = la + ba_ref[...]
  la = la - jnp.max(la, axis=-1, keepdims=True)
  e = jnp.exp(la)
  den = lax.dot_general(e, s4_ref[...], dn_n, preferred_element_type=_f32)
  attn = lax.dot_general(e / den, sa_ref[...], dn_n,
                         preferred_element_type=_f32)

  qx3 = lax.broadcasted_iota(_i32, (4, W, 128), 1)
  qr3 = lax.broadcasted_iota(_i32, (4, W, 128), 0)
  qx = qx3.reshape(T, 128).astype(_f32)
  qy = ((i % NBLK) * 4 + qr3.reshape(T, 128)).astype(_f32)

  lane = lax.broadcasted_iota(_i32, (T, 128), 1)
  cx = (lane % 2).astype(_f32)
  cy = ((lane % 4) // 2).astype(_f32)
  px = qx + offx
  py = qy + offy
  x0 = jnp.floor(px)
  y0 = jnp.floor(py)
  fx = px - x0
  fy = py - y0
  xi = x0 + cx
  yi = y0 + cy
  wx = cx * fx + (1.0 - cx) * (1.0 - fx)
  wy = cy * fy + (1.0 - cy) * (1.0 - fy)
  valid = ((xi >= 0.0) & (xi <= W - 1.0) & (yi >= 0.0) & (yi <= H - 1.0))
  wt_ref[...] = wx * wy * attn * valid.astype(_f32)
  xi_i = jnp.clip(xi, 0.0, W - 1.0).astype(_i32)
  yi_i = jnp.clip(yi, 0.0, H - 1.0).astype(_i32)
  b = i // NBLK
  idx_ref[...] = (b * LEN + yi_i * W + xi_i) * NH + lane // 16


def _post_body(s_ref, wo_ref, bo_ref, out_ref):
  dn = (((1,), (0,)), ((), ()))
  out_ref[0] = (lax.dot_general(s_ref[...], wo_ref[...], dn,
                                preferred_element_type=_f32) + bo_ref[...])


def _sc_sample(table, idx2d, wtflat):
  mesh = plsc.VectorSubcoreMesh(core_axis_name="c", subcore_axis_name="s")

  @functools.partial(
      pl.kernel, mesh=mesh,
      compiler_params=pltpu.CompilerParams(use_tc_tiling_on_sc=False),
      out_type=jax.ShapeDtypeStruct((NGROUP, DH), _f32),
      scratch_types=[
          pltpu.VMEM((KS, 128), _i32),
          pltpu.VMEM((KS, 128), _i32),
          pltpu.VMEM((CG, DH), _f32),
          pltpu.VMEM((CG, DH), _f32),
          pltpu.VMEM((CG * DH, DH), _f32),
          pltpu.VMEM((CG * DH, DH), _f32),
          pltpu.VMEM((CG, DH), _f32),
          pltpu.VMEM((CG, DH), _f32),
          pltpu.SemaphoreType.DMA,
          pltpu.SemaphoreType.DMA,
          pltpu.SemaphoreType.DMA,
          pltpu.SemaphoreType.DMA,
      ],
  )
  def k(table_hbm, idx_hbm, wt_hbm, out_hbm, idx_a, idx_b, wt_a, wt_b,
        rows_a, rows_b, out_a, out_b, sem_a, sem_b, sem_oa, sem_ob):
    wid = lax.axis_index("s") * 2 + lax.axis_index("c")

    def fire_idx(ci, idx_v, sem):
      gbase = pl.multiple_of(wid * GPW + ci * CG, CG)
      pltpu.async_copy(
          idx_hbm.at[pl.ds(pl.multiple_of(gbase * DH // 128, 8), KS)], idx_v,
          sem)

    def fire_wt(ci, wt_v, sem):
      gbase = pl.multiple_of(wid * GPW + ci * CG, CG)
      pltpu.async_copy(wt_hbm.at[pl.ds(gbase, CG)], wt_v, sem)

    def fire_gathers(idx_v, rows_v, sem):
      pltpu.make_async_copy(
          idx_hbm.at[pl.ds(0, KS)], idx_v, sem).wait()
      for s in range(KS):
        pltpu.async_copy(table_hbm.at[idx_v.at[s]],
                         rows_v.at[pl.ds(s * 128, 128)], sem)

    def compute(ci, idx_v, wt_v, rows_v, sem, out_v, sem_o):
      gbase = pl.multiple_of(wid * GPW + ci * CG, CG)
      pltpu.make_async_copy(table_hbm.at[pl.ds(0, CG * DH)], rows_v,
                            sem).wait()
      pltpu.make_async_copy(wt_hbm.at[pl.ds(0, CG)], wt_v, sem).wait()

      @pl.when(ci + 2 < NCHUNK)
      def _():
        fire_idx(ci + 2, idx_v, sem)

      @pl.when(ci >= 2)
      def _():
        pltpu.make_async_copy(out_v, out_hbm.at[pl.ds(0, CG)], sem_o).wait()

      @plsc.parallel_loop(0, CG, unroll=16)
      def group_body(g):
        wb = g * DH
        w_vec = wt_v[g]
        acc = jnp.zeros((DH,), _f32)
        for j in range(DH):
          wj = lax.gather(
              w_vec, jnp.full((DH, 1), j, _i32),
              lax.GatherDimensionNumbers(offset_dims=(),
                                         collapsed_slice_dims=(0,),
                                         start_index_map=(0,)),
              slice_sizes=(1,),
              mode=lax.GatherScatterMode.PROMISE_IN_BOUNDS)
          acc = acc + wj * rows_v[wb + j]
        out_v[g] = acc

      @pl.when(ci + 2 < NCHUNK)
      def _():
        fire_wt(ci + 2, wt_v, sem)

      pltpu.async_copy(out_v, out_hbm.at[pl.ds(gbase, CG)], sem_o)

    fire_idx(0, idx_a, sem_a)
    fire_wt(0, wt_a, sem_a)
    fire_idx(1, idx_b, sem_b)
    fire_wt(1, wt_b, sem_b)
    fire_gathers(idx_a, rows_a, sem_a)

    def pair_body(t, _):
      fire_gathers(idx_b, rows_b, sem_b)
      compute(2 * t, idx_a, wt_a, rows_a, sem_a, out_a, sem_oa)

      @pl.when(t < NCHUNK // 2 - 1)
      def _():
        fire_gathers(idx_a, rows_a, sem_a)

      compute(2 * t + 1, idx_b, wt_b, rows_b, sem_b, out_b, sem_ob)
      return 0

    lax.fori_loop(0, NCHUNK // 2, pair_body, 0)
    pltpu.make_async_copy(out_a, out_hbm.at[pl.ds(0, CG)], sem_oa).wait()
    pltpu.make_async_copy(out_b, out_hbm.at[pl.ds(0, CG)], sem_ob).wait()

  return k(table, idx2d, wtflat)


def kernel(nbr_fea, ext_fea, W_value, b_value, W_off, b_off, W_attn, b_attn,
           W_out, b_out):
  sx, sy, sa, s4 = _selection_mats()
  wox = W_off @ sx
  box = (b_off @ sx).reshape(1, 128)
  woy = W_off @ sy
  boy = (b_off @ sy).reshape(1, 128)

  nbr = nbr_fea.reshape(B, D, LEN)
  ext = ext_fea.reshape(B, D, LEN)
  full = lambda s: pl.BlockSpec(s, lambda i: (0,) * len(s))
  val, idx, wt = pl.pallas_call(
      _prep_body,
      grid=(GRID,),
      in_specs=[
          pl.BlockSpec((1, D, T), lambda i: (i // NBLK, 0, i % NBLK)),
          pl.BlockSpec((1, D, T), lambda i: (i // NBLK, 0, i % NBLK)),
          full((D, D)), full((1, D)),
          full((D, 128)), full((1, 128)),
          full((D, 128)), full((1, 128)),
          full((D, NH * P)), full((1, NH * P)),
          full((NH * P, NH * P)), full((NH * P, 128)),
      ],
      out_specs=[
          pl.BlockSpec((T, D), lambda i: (i, 0)),
          pl.BlockSpec((T, 128), lambda i: (i, 0)),
          pl.BlockSpec((T, 128), lambda i: (i, 0)),
      ],
      out_shape=[
          jax.ShapeDtypeStruct((B * LEN, D), _f32),
          jax.ShapeDtypeStruct((B * LEN, 128), _i32),
          jax.ShapeDtypeStruct((B * LEN, 128), _f32),
      ],
  )(nbr, ext, W_value, b_value.reshape(1, D), wox, box, woy, boy,
    W_attn, b_attn.reshape(1, NH * P), s4, sa)

  sampled = _sc_sample(val.reshape(NGROUP, DH),
                       idx.reshape(IDX_ROWS, 128),
                       wt.reshape(NGROUP, DH))

  out = pl.pallas_call(
      _post_body,
      grid=(GRID,),
      in_specs=[
          pl.BlockSpec((T, D), lambda i: (i, 0)),
          full((D, D)), full((1, D)),
      ],
      out_specs=pl.BlockSpec((1, T, D), lambda i: (i // NBLK, i % NBLK, 0)),
      out_shape=jax.ShapeDtypeStruct((B, LEN, D), _f32),
  )(sampled.reshape(B * LEN, D), W_out, b_out.reshape(1, D))
  return out
